# baseline (speedup 1.0000x reference)
import jax
import jax.numpy as jnp
from jax import lax
from jax.experimental import pallas as pl
from jax.experimental.pallas import tpu as pltpu
from jax.experimental.pallas import tpu_sc as plsc

N = 10000
E = 320000
D = 128

NC = 2
NS = 16
NW = NC * NS
EPW = E // NW
B = 40
CHUNKS = EPW // B
GDEPTH = 4
SLAG = 2
NBUF = GDEPTH + SLAG
IDEPTH = GDEPTH + 1
NIBUF = IDEPTH + SLAG
ROWS_PER_TILE = N // NS
ZR = 25
LANES = 16


def _spmm_body(idx_hbm, embeds_hbm, out_hbm,
               idxb, rows, stage_v, acc, gsem, ssem, isem, zsem):
    cid = lax.axis_index("c")
    sid = lax.axis_index("s")
    wid = sid * NC + cid

    def _zero_row(i, c):
        for j in range(D // LANES):
            stage_v[i, pl.ds(j * LANES, LANES)] = jnp.zeros((LANES,), jnp.float32)
        return c
    lax.fori_loop(0, ZR, _zero_row, 0)
    for k in range(ROWS_PER_TILE // ZR):
        pltpu.async_copy(
            stage_v, acc.at[pl.ds(sid * ROWS_PER_TILE + k * ZR, ZR), :], zsem)
    for k in range(ROWS_PER_TILE // ZR):
        pltpu.make_async_copy(
            stage_v, acc.at[pl.ds(sid * ROWS_PER_TILE + k * ZR, ZR), :],
            zsem).wait()
    plsc.subcore_barrier()

    def _idx_fetch_start(ci):
        ib = lax.rem(ci, NIBUF)
        pltpu.async_copy(idx_hbm.at[wid, ci], idxb.at[ib], isem.at[ib])

    def _idx_fetch_wait(ci):
        ib = lax.rem(ci, NIBUF)
        pltpu.make_async_copy(idx_hbm.at[wid, ci], idxb.at[ib],
                              isem.at[ib]).wait()

    def _gather_start(ci):
        ib, b = lax.rem(ci, NIBUF), lax.rem(ci, NBUF)
        pltpu.async_copy(embeds_hbm.at[idxb.at[ib, 1]], rows.at[b], gsem.at[b])

    def _gather_wait(ci):
        ib, b = lax.rem(ci, NIBUF), lax.rem(ci, NBUF)
        pltpu.make_async_copy(embeds_hbm.at[idxb.at[ib, 1]], rows.at[b],
                              gsem.at[b]).wait()

    def _scatter_start(ci):
        ib, b = lax.rem(ci, NIBUF), lax.rem(ci, NBUF)
        pltpu.async_copy(rows.at[b], acc.at[idxb.at[ib, 0]], ssem.at[b],
                         add=True)

    def _scatter_wait(ci):
        ib, b = lax.rem(ci, NIBUF), lax.rem(ci, NBUF)
        pltpu.make_async_copy(rows.at[b], acc.at[idxb.at[ib, 0]],
                              ssem.at[b]).wait()

    for k in range(IDEPTH):
        _idx_fetch_start(k)
    for k in range(GDEPTH):
        _idx_fetch_wait(k)
        _gather_start(k)

    def _chunk(ci, c):
        @pl.when(ci >= SLAG)
        def _():
            _scatter_wait(ci - SLAG)

        @pl.when(ci + IDEPTH < CHUNKS)
        def _():
            _idx_fetch_start(ci + IDEPTH)

        @pl.when(ci + GDEPTH < CHUNKS)
        def _():
            _idx_fetch_wait(ci + GDEPTH)
            _gather_start(ci + GDEPTH)

        _gather_wait(ci)

        rb = rows.at[lax.rem(ci, NBUF)]
        vb = lax.rem(ci, NIBUF)

        @plsc.parallel_loop(0, B, unroll=8)
        def _edge(e):
            vbits = plsc.load_gather(
                idxb, [jnp.full((LANES,), vb, jnp.int32),
                       jnp.full((LANES,), 2, jnp.int32),
                       jnp.full((LANES,), e, jnp.int32)])
            ve = plsc.bitcast(vbits, jnp.float32)
            for j in range(D // LANES):
                sl = pl.ds(j * LANES, LANES)
                rb[e, sl] = rb[e, sl] * ve

        _scatter_start(ci)
        return c
    lax.fori_loop(0, CHUNKS, _chunk, 0)

    for k in range(SLAG):
        _scatter_wait(CHUNKS - SLAG + k)
    plsc.subcore_barrier()

    for k in range(ROWS_PER_TILE // ZR):
        b0 = sid * ROWS_PER_TILE + k * ZR
        pltpu.async_copy(acc.at[pl.ds(b0, ZR), :],
                         out_hbm.at[cid, pl.ds(b0, ZR), :], zsem)
    for k in range(ROWS_PER_TILE // ZR):
        b0 = sid * ROWS_PER_TILE + k * ZR
        pltpu.make_async_copy(acc.at[pl.ds(b0, ZR), :],
                              out_hbm.at[cid, pl.ds(b0, ZR), :], zsem).wait()


_spmm_sc = pl.kernel(
    _spmm_body,
    out_type=jax.ShapeDtypeStruct((NC, N, D), jnp.float32),
    mesh=plsc.VectorSubcoreMesh(core_axis_name="c", subcore_axis_name="s",
                                num_cores=NC, num_subcores=NS),
    compiler_params=pltpu.CompilerParams(use_tc_tiling_on_sc=False,
                                         needs_layout_passes=False),
    scratch_types=[
        pltpu.VMEM((NIBUF, 3, B), jnp.int32),
        pltpu.VMEM((NBUF, B, D), jnp.float32),
        pltpu.VMEM((ZR, D), jnp.float32),
        pltpu.VMEM_SHARED((N, D), jnp.float32),
        pltpu.SemaphoreType.DMA((NBUF,)),
        pltpu.SemaphoreType.DMA((NBUF,)),
        pltpu.SemaphoreType.DMA((NIBUF,)),
        pltpu.SemaphoreType.DMA,
    ],
)


def _add_body(a_ref, b_ref, o_ref):
    o_ref[...] = a_ref[...] + b_ref[...]


def _combine(p0, p1):
    blk = 1000
    return pl.pallas_call(
        _add_body,
        out_shape=jax.ShapeDtypeStruct((N, D), jnp.float32),
        grid=(N // blk,),
        in_specs=[pl.BlockSpec((blk, D), lambda i: (i, 0))] * 2,
        out_specs=pl.BlockSpec((blk, D), lambda i: (i, 0)),
    )(p0, p1)


@jax.jit
def kernel(adj_indices, adj_values, embeds):
    row = adj_indices[0].reshape(NW, CHUNKS, 1, B)
    col = adj_indices[1].reshape(NW, CHUNKS, 1, B)
    vbits = lax.bitcast_convert_type(adj_values, jnp.int32)
    vbits = vbits.reshape(NW, CHUNKS, 1, B)
    idx = jnp.concatenate([row, col, vbits], axis=2)
    partials = _spmm_sc(idx, embeds)
    return _combine(partials[0], partials[1])


# speedup vs baseline: 1.3912x; 1.3912x over previous
import jax
import jax.numpy as jnp
from jax import lax
from jax.experimental import pallas as pl
from jax.experimental.pallas import tpu as pltpu
from jax.experimental.pallas import tpu_sc as plsc

N = 10000
E = 320000
D = 128

NC = 2
NS = 16
NW = NC * NS
EPW = E // NW
B = 40
CHUNKS = EPW // B
GDEPTH = 4
SLAG = 2
NBUF = GDEPTH + SLAG
IDEPTH = GDEPTH + 1
NIBUF = IDEPTH + SLAG
ROWS_PER_TILE = N // NS
ZR = 25
LANES = 16


def _spmm_body(row_hbm, col_hbm, val_hbm, embeds_hbm, out_hbm,
               valb, rowb, colb, rows, stage_v, acc, gsem, ssem, isem, zsem):
    cid = lax.axis_index("c")
    sid = lax.axis_index("s")
    wid = sid * NC + cid

    def _zero_row(i, c):
        for j in range(D // LANES):
            stage_v[i, pl.ds(j * LANES, LANES)] = jnp.zeros((LANES,), jnp.float32)
        return c
    lax.fori_loop(0, ZR, _zero_row, 0)
    for k in range(ROWS_PER_TILE // ZR):
        pltpu.async_copy(
            stage_v, acc.at[pl.ds(sid * ROWS_PER_TILE + k * ZR, ZR), :], zsem)
    for k in range(ROWS_PER_TILE // ZR):
        pltpu.make_async_copy(
            stage_v, acc.at[pl.ds(sid * ROWS_PER_TILE + k * ZR, ZR), :],
            zsem).wait()
    plsc.subcore_barrier()

    def _idx_fetch_start(ci):
        ib = lax.rem(ci, NIBUF)
        pltpu.async_copy(col_hbm.at[wid, ci], colb.at[ib], isem.at[ib])
        pltpu.async_copy(row_hbm.at[wid, ci], rowb.at[ib], isem.at[ib])
        pltpu.async_copy(val_hbm.at[wid, ci], valb.at[ib], isem.at[ib])

    def _idx_fetch_wait(ci):
        ib = lax.rem(ci, NIBUF)
        pltpu.make_async_copy(col_hbm.at[wid, ci], colb.at[ib],
                              isem.at[ib]).wait()
        pltpu.make_async_copy(row_hbm.at[wid, ci], rowb.at[ib],
                              isem.at[ib]).wait()
        pltpu.make_async_copy(val_hbm.at[wid, ci], valb.at[ib],
                              isem.at[ib]).wait()

    def _gather_start(ci):
        ib, b = lax.rem(ci, NIBUF), lax.rem(ci, NBUF)
        pltpu.async_copy(embeds_hbm.at[colb.at[ib]], rows.at[b], gsem.at[b])

    def _gather_wait(ci):
        ib, b = lax.rem(ci, NIBUF), lax.rem(ci, NBUF)
        pltpu.make_async_copy(embeds_hbm.at[colb.at[ib]], rows.at[b],
                              gsem.at[b]).wait()

    def _scatter_start(ci):
        ib, b = lax.rem(ci, NIBUF), lax.rem(ci, NBUF)
        pltpu.async_copy(rows.at[b], acc.at[rowb.at[ib]], ssem.at[b], add=True)

    def _scatter_wait(ci):
        ib, b = lax.rem(ci, NIBUF), lax.rem(ci, NBUF)
        pltpu.make_async_copy(rows.at[b], acc.at[rowb.at[ib]],
                              ssem.at[b]).wait()

    for k in range(IDEPTH):
        _idx_fetch_start(k)
    for k in range(GDEPTH):
        _idx_fetch_wait(k)
        _gather_start(k)

    def _chunk(ci, c):
        b = lax.rem(ci, NBUF)

        @pl.when(ci >= SLAG)
        def _():
            _scatter_wait(ci - SLAG)

        @pl.when(ci + IDEPTH < CHUNKS)
        def _():
            _idx_fetch_start(ci + IDEPTH)

        @pl.when(ci + GDEPTH < CHUNKS)
        def _():
            _idx_fetch_wait(ci + GDEPTH)
            _gather_start(ci + GDEPTH)

        _gather_wait(ci)

        rb = rows.at[b]
        vb = lax.rem(ci, NIBUF)

        @plsc.parallel_loop(0, B, unroll=8)
        def _edge(e):
            ve = plsc.load_gather(
                valb, [jnp.full((LANES,), vb, jnp.int32),
                       jnp.full((LANES,), e, jnp.int32)])
            for j in range(D // LANES):
                sl = pl.ds(j * LANES, LANES)
                rb[e, sl] = rb[e, sl] * ve

        _scatter_start(ci)
        return c
    lax.fori_loop(0, CHUNKS, _chunk, 0)

    for k in range(SLAG):
        _scatter_wait(CHUNKS - SLAG + k)
    plsc.subcore_barrier()

    for k in range(ROWS_PER_TILE // ZR):
        b0 = sid * ROWS_PER_TILE + k * ZR
        pltpu.async_copy(acc.at[pl.ds(b0, ZR), :],
                         out_hbm.at[cid, pl.ds(b0, ZR), :], zsem)
    for k in range(ROWS_PER_TILE // ZR):
        b0 = sid * ROWS_PER_TILE + k * ZR
        pltpu.make_async_copy(acc.at[pl.ds(b0, ZR), :],
                              out_hbm.at[cid, pl.ds(b0, ZR), :], zsem).wait()


_spmm_sc = pl.kernel(
    _spmm_body,
    out_type=jax.ShapeDtypeStruct((NC, N, D), jnp.float32),
    mesh=plsc.VectorSubcoreMesh(core_axis_name="c", subcore_axis_name="s",
                                num_cores=NC, num_subcores=NS),
    compiler_params=pltpu.CompilerParams(use_tc_tiling_on_sc=False,
                                         needs_layout_passes=False),
    scratch_types=[
        pltpu.VMEM((NIBUF, B), jnp.float32),
        pltpu.VMEM((NIBUF, B), jnp.int32),
        pltpu.VMEM((NIBUF, B), jnp.int32),
        pltpu.VMEM((NBUF, B, D), jnp.float32),
        pltpu.VMEM((ZR, D), jnp.float32),
        pltpu.VMEM_SHARED((N, D), jnp.float32),
        pltpu.SemaphoreType.DMA((NBUF,)),
        pltpu.SemaphoreType.DMA((NBUF,)),
        pltpu.SemaphoreType.DMA((NIBUF,)),
        pltpu.SemaphoreType.DMA,
    ],
)


def _add_body(a_ref, b_ref, o_ref):
    o_ref[...] = a_ref[...] + b_ref[...]


def _combine(p0, p1):
    blk = 1000
    return pl.pallas_call(
        _add_body,
        out_shape=jax.ShapeDtypeStruct((N, D), jnp.float32),
        grid=(N // blk,),
        in_specs=[pl.BlockSpec((blk, D), lambda i: (i, 0))] * 2,
        out_specs=pl.BlockSpec((blk, D), lambda i: (i, 0)),
    )(p0, p1)


@jax.jit
def kernel(adj_indices, adj_values, embeds):
    row = adj_indices[0].reshape(NW, CHUNKS, B)
    col = adj_indices[1].reshape(NW, CHUNKS, B)
    val = adj_values.reshape(NW, CHUNKS, B)
    partials = _spmm_sc(row, col, val, embeds)
    return _combine(partials[0], partials[1])


# speedup vs baseline: 1.4118x; 1.0148x over previous
import jax
import jax.numpy as jnp
from jax import lax
from jax.experimental import pallas as pl
from jax.experimental.pallas import tpu as pltpu
from jax.experimental.pallas import tpu_sc as plsc

N = 10000
E = 320000
D = 128

NC = 2
NS = 16
NW = NC * NS
EPW = E // NW
B = 40
CHUNKS = EPW // B
GDEPTH = 4
SLAG = 2
NBUF = GDEPTH + SLAG
IDEPTH = GDEPTH + 1
NIBUF = IDEPTH + SLAG
ROWS_PER_TILE = N // NS
ZR = 25
LANES = 16


def _spmm_body(row_hbm, col_hbm, val_hbm, embeds_hbm, out_hbm,
               valb, rowb, colb, rows, stage_v, acc, gsem, ssem, isem, zsem):
    cid = lax.axis_index("c")
    sid = lax.axis_index("s")
    wid = sid * NC + cid

    def _zero_row(i, c):
        for j in range(D // LANES):
            stage_v[i, pl.ds(j * LANES, LANES)] = jnp.zeros((LANES,), jnp.float32)
        return c
    lax.fori_loop(0, ZR, _zero_row, 0)
    for k in range(ROWS_PER_TILE // ZR):
        pltpu.async_copy(
            stage_v, acc.at[pl.ds(sid * ROWS_PER_TILE + k * ZR, ZR), :], zsem)
    for k in range(ROWS_PER_TILE // ZR):
        pltpu.make_async_copy(
            stage_v, acc.at[pl.ds(sid * ROWS_PER_TILE + k * ZR, ZR), :],
            zsem).wait()
    plsc.subcore_barrier()

    def _idx_fetch_start(ci):
        ib = lax.rem(ci, NIBUF)
        pltpu.async_copy(col_hbm.at[wid, ci], colb.at[ib], isem.at[ib])
        pltpu.async_copy(row_hbm.at[wid, ci], rowb.at[ib], isem.at[ib])
        pltpu.async_copy(val_hbm.at[wid, ci], valb.at[ib], isem.at[ib])

    def _idx_fetch_wait(ci):
        ib = lax.rem(ci, NIBUF)
        pltpu.make_async_copy(col_hbm.at[wid, ci], colb.at[ib],
                              isem.at[ib]).wait()
        pltpu.make_async_copy(row_hbm.at[wid, ci], rowb.at[ib],
                              isem.at[ib]).wait()
        pltpu.make_async_copy(val_hbm.at[wid, ci], valb.at[ib],
                              isem.at[ib]).wait()

    def _gather_start(ci):
        ib, b = lax.rem(ci, NIBUF), lax.rem(ci, NBUF)
        pltpu.async_copy(embeds_hbm.at[colb.at[ib]], rows.at[b], gsem.at[b])

    def _gather_wait(ci):
        ib, b = lax.rem(ci, NIBUF), lax.rem(ci, NBUF)
        pltpu.make_async_copy(embeds_hbm.at[colb.at[ib]], rows.at[b],
                              gsem.at[b]).wait()

    def _scatter_start(ci):
        ib, b = lax.rem(ci, NIBUF), lax.rem(ci, NBUF)
        pltpu.async_copy(rows.at[b], acc.at[rowb.at[ib]], ssem.at[b], add=True)

    def _scatter_wait(ci):
        ib, b = lax.rem(ci, NIBUF), lax.rem(ci, NBUF)
        pltpu.make_async_copy(rows.at[b], acc.at[rowb.at[ib]],
                              ssem.at[b]).wait()

    for k in range(IDEPTH):
        _idx_fetch_start(k)
    for k in range(GDEPTH):
        _idx_fetch_wait(k)
        _gather_start(k)

    def _chunk(ci, c):
        b = lax.rem(ci, NBUF)

        @pl.when(ci >= SLAG)
        def _():
            _scatter_wait(ci - SLAG)

        @pl.when(ci + IDEPTH < CHUNKS)
        def _():
            _idx_fetch_start(ci + IDEPTH)

        @pl.when(ci + GDEPTH < CHUNKS)
        def _():
            _idx_fetch_wait(ci + GDEPTH)
            _gather_start(ci + GDEPTH)

        _gather_wait(ci)

        rb = rows.at[b]
        vb = lax.rem(ci, NIBUF)

        @plsc.parallel_loop(0, B, unroll=8)
        def _edge(e):
            ve = plsc.load_gather(
                valb, [jnp.full((LANES,), vb, jnp.int32),
                       jnp.full((LANES,), e, jnp.int32)])
            for j in range(D // LANES):
                sl = pl.ds(j * LANES, LANES)
                rb[e, sl] = rb[e, sl] * ve

        _scatter_start(ci)
        return c
    lax.fori_loop(0, CHUNKS, _chunk, 0)

    for k in range(SLAG):
        _scatter_wait(CHUNKS - SLAG + k)
    plsc.subcore_barrier()

    for k in range(ROWS_PER_TILE // ZR):
        b0 = sid * ROWS_PER_TILE + k * ZR
        pltpu.async_copy(acc.at[pl.ds(b0, ZR), :],
                         out_hbm.at[cid, pl.ds(b0, ZR), :], zsem)
    for k in range(ROWS_PER_TILE // ZR):
        b0 = sid * ROWS_PER_TILE + k * ZR
        pltpu.make_async_copy(acc.at[pl.ds(b0, ZR), :],
                              out_hbm.at[cid, pl.ds(b0, ZR), :], zsem).wait()


_spmm_sc = pl.kernel(
    _spmm_body,
    out_type=jax.ShapeDtypeStruct((NC, N, D), jnp.float32),
    mesh=plsc.VectorSubcoreMesh(core_axis_name="c", subcore_axis_name="s",
                                num_cores=NC, num_subcores=NS),
    compiler_params=pltpu.CompilerParams(use_tc_tiling_on_sc=False,
                                         needs_layout_passes=False),
    scratch_types=[
        pltpu.VMEM((NIBUF, B), jnp.float32),
        pltpu.VMEM((NIBUF, B), jnp.int32),
        pltpu.VMEM((NIBUF, B), jnp.int32),
        pltpu.VMEM((NBUF, B, D), jnp.float32),
        pltpu.VMEM((ZR, D), jnp.float32),
        pltpu.VMEM_SHARED((N, D), jnp.float32),
        pltpu.SemaphoreType.DMA((NBUF,)),
        pltpu.SemaphoreType.DMA((NBUF,)),
        pltpu.SemaphoreType.DMA((NIBUF,)),
        pltpu.SemaphoreType.DMA,
    ],
)


def _add_body(a_ref, b_ref, o_ref):
    o_ref[...] = a_ref[...] + b_ref[...]


def _combine(p0, p1):
    blk = 2000
    return pl.pallas_call(
        _add_body,
        out_shape=jax.ShapeDtypeStruct((N, D), jnp.float32),
        grid=(N // blk,),
        in_specs=[pl.BlockSpec((blk, D), lambda i: (i, 0))] * 2,
        out_specs=pl.BlockSpec((blk, D), lambda i: (i, 0)),
    )(p0, p1)


@jax.jit
def kernel(adj_indices, adj_values, embeds):
    row = adj_indices[0].reshape(NW, CHUNKS, B)
    col = adj_indices[1].reshape(NW, CHUNKS, B)
    val = adj_values.reshape(NW, CHUNKS, B)
    partials = _spmm_sc(row, col, val, embeds)
    return _combine(partials[0], partials[1])


# speedup vs baseline: 1.4265x; 1.0104x over previous
import jax
import jax.numpy as jnp
from jax import lax
from jax.experimental import pallas as pl
from jax.experimental.pallas import tpu as pltpu
from jax.experimental.pallas import tpu_sc as plsc

N = 10000
E = 320000
D = 128

NC = 2
NS = 16
NW = NC * NS
EPW = E // NW
B = 40
CHUNKS = EPW // B
GDEPTH = 4
SLAG = 2
NBUF = GDEPTH + SLAG
IDEPTH = GDEPTH + 1
NIBUF = IDEPTH + SLAG
ROWS_PER_TILE = N // NS
ZR = 25
LANES = 16


def _spmm_body(row_hbm, col_hbm, val_hbm, embeds_hbm, out_hbm,
               valb, rowb, colb, rows, stage_v, acc, gsem, ssem, isem, zsem):
    cid = lax.axis_index("c")
    sid = lax.axis_index("s")
    wid = sid * NC + cid

    def _zero_row(i, c):
        for j in range(D // LANES):
            stage_v[i, pl.ds(j * LANES, LANES)] = jnp.zeros((LANES,), jnp.float32)
        return c
    lax.fori_loop(0, ZR, _zero_row, 0)
    for k in range(ROWS_PER_TILE // ZR):
        pltpu.async_copy(
            stage_v, acc.at[pl.ds(sid * ROWS_PER_TILE + k * ZR, ZR), :], zsem)
    for k in range(ROWS_PER_TILE // ZR):
        pltpu.make_async_copy(
            stage_v, acc.at[pl.ds(sid * ROWS_PER_TILE + k * ZR, ZR), :],
            zsem).wait()
    plsc.subcore_barrier()

    def _idx_fetch_start(ci):
        ib = lax.rem(ci, NIBUF)
        pltpu.async_copy(col_hbm.at[wid, ci], colb.at[ib], isem.at[ib])
        pltpu.async_copy(row_hbm.at[wid, ci], rowb.at[ib], isem.at[ib])
        pltpu.async_copy(val_hbm.at[wid, ci], valb.at[ib], isem.at[ib])

    def _idx_fetch_wait(ci):
        ib = lax.rem(ci, NIBUF)
        pltpu.make_async_copy(col_hbm.at[wid, ci], colb.at[ib],
                              isem.at[ib]).wait()
        pltpu.make_async_copy(row_hbm.at[wid, ci], rowb.at[ib],
                              isem.at[ib]).wait()
        pltpu.make_async_copy(val_hbm.at[wid, ci], valb.at[ib],
                              isem.at[ib]).wait()

    def _gather_start(ci):
        ib, b = lax.rem(ci, NIBUF), lax.rem(ci, NBUF)
        pltpu.async_copy(embeds_hbm.at[colb.at[ib]], rows.at[b], gsem.at[b])

    def _gather_wait(ci):
        ib, b = lax.rem(ci, NIBUF), lax.rem(ci, NBUF)
        pltpu.make_async_copy(embeds_hbm.at[colb.at[ib]], rows.at[b],
                              gsem.at[b]).wait()

    def _scatter_start(ci):
        ib, b = lax.rem(ci, NIBUF), lax.rem(ci, NBUF)
        pltpu.async_copy(rows.at[b], acc.at[rowb.at[ib]], ssem.at[b], add=True)

    def _scatter_wait(ci):
        ib, b = lax.rem(ci, NIBUF), lax.rem(ci, NBUF)
        pltpu.make_async_copy(rows.at[b], acc.at[rowb.at[ib]],
                              ssem.at[b]).wait()

    for k in range(IDEPTH):
        _idx_fetch_start(k)
    for k in range(GDEPTH):
        _idx_fetch_wait(k)
        _gather_start(k)

    def _chunk(ci, c):
        b = lax.rem(ci, NBUF)

        @pl.when(ci >= SLAG)
        def _():
            _scatter_wait(ci - SLAG)

        @pl.when(ci + IDEPTH < CHUNKS)
        def _():
            _idx_fetch_start(ci + IDEPTH)

        @pl.when(ci + GDEPTH < CHUNKS)
        def _():
            _idx_fetch_wait(ci + GDEPTH)
            _gather_start(ci + GDEPTH)

        _gather_wait(ci)

        rb = rows.at[b]
        vb = lax.rem(ci, NIBUF)

        @plsc.parallel_loop(0, B, unroll=8)
        def _edge(e):
            ve = plsc.load_gather(
                valb, [jnp.full((LANES,), vb, jnp.int32),
                       jnp.full((LANES,), e, jnp.int32)])
            for j in range(D // LANES):
                sl = pl.ds(j * LANES, LANES)
                rb[e, sl] = rb[e, sl] * ve

        _scatter_start(ci)
        return c
    lax.fori_loop(0, CHUNKS, _chunk, 0)

    for k in range(SLAG):
        _scatter_wait(CHUNKS - SLAG + k)
    plsc.subcore_barrier()

    for k in range(ROWS_PER_TILE // ZR):
        b0 = sid * ROWS_PER_TILE + k * ZR
        pltpu.async_copy(acc.at[pl.ds(b0, ZR), :],
                         out_hbm.at[cid, pl.ds(b0, ZR), :], zsem)
    for k in range(ROWS_PER_TILE // ZR):
        b0 = sid * ROWS_PER_TILE + k * ZR
        pltpu.make_async_copy(acc.at[pl.ds(b0, ZR), :],
                              out_hbm.at[cid, pl.ds(b0, ZR), :], zsem).wait()


_spmm_sc = pl.kernel(
    _spmm_body,
    out_type=jax.ShapeDtypeStruct((NC, N, D), jnp.float32),
    mesh=plsc.VectorSubcoreMesh(core_axis_name="c", subcore_axis_name="s",
                                num_cores=NC, num_subcores=NS),
    compiler_params=pltpu.CompilerParams(use_tc_tiling_on_sc=False,
                                         needs_layout_passes=False),
    scratch_types=[
        pltpu.VMEM((NIBUF, B), jnp.float32),
        pltpu.VMEM((NIBUF, B), jnp.int32),
        pltpu.VMEM((NIBUF, B), jnp.int32),
        pltpu.VMEM((NBUF, B, D), jnp.float32),
        pltpu.VMEM((ZR, D), jnp.float32),
        pltpu.VMEM_SHARED((N, D), jnp.float32),
        pltpu.SemaphoreType.DMA((NBUF,)),
        pltpu.SemaphoreType.DMA((NBUF,)),
        pltpu.SemaphoreType.DMA((NIBUF,)),
        pltpu.SemaphoreType.DMA,
    ],
)


def _add_body(a_ref, b_ref, o_ref):
    o_ref[...] = a_ref[...] + b_ref[...]


def _combine(p0, p1):
    blk = 5000
    return pl.pallas_call(
        _add_body,
        out_shape=jax.ShapeDtypeStruct((N, D), jnp.float32),
        grid=(N // blk,),
        in_specs=[pl.BlockSpec((blk, D), lambda i: (i, 0))] * 2,
        out_specs=pl.BlockSpec((blk, D), lambda i: (i, 0)),
    )(p0, p1)


@jax.jit
def kernel(adj_indices, adj_values, embeds):
    row = adj_indices[0].reshape(NW, CHUNKS, B)
    col = adj_indices[1].reshape(NW, CHUNKS, B)
    val = adj_values.reshape(NW, CHUNKS, B)
    partials = _spmm_sc(row, col, val, embeds)
    return _combine(partials[0], partials[1])
